# baseline (speedup 1.0000x reference)
import functools

import jax
import jax.numpy as jnp
from jax import lax
from jax.experimental import pallas as pl
from jax.experimental.pallas import tpu as pltpu
from jax.experimental.pallas import tpu_sc as plsc

NC = 2
NS = 16
NW = NC * NS
K = 80
KD = 400


def _deg_body(np_, nb, src_hbm, dst_hbm, ones_s_hbm, ones_d_hbm, zeros_hbm,
              out_hbm, idx_s, idx_d, ones_s, ones_d, stage, deg_sh,
              sem_s, sem_d):
    cid = lax.axis_index("c")
    sid = lax.axis_index("s")
    wid = sid * NC + cid
    rpt = np_ // NS
    pltpu.sync_copy(src_hbm.at[wid], idx_s)
    pltpu.sync_copy(dst_hbm.at[wid], idx_d)
    pltpu.sync_copy(ones_s_hbm, ones_s)
    pltpu.sync_copy(ones_d_hbm, ones_d)
    pltpu.sync_copy(zeros_hbm, stage)
    pltpu.sync_copy(stage, deg_sh.at[pl.ds(sid * rpt, rpt)])
    plsc.subcore_barrier()

    def body(j, c):
        pltpu.async_copy(ones_s, deg_sh.at[idx_s.at[j]], sem_s, add=True)
        pltpu.async_copy(ones_d, deg_sh.at[idx_d.at[j]], sem_d, add=True)
        return c

    lax.fori_loop(0, nb, body, 0)

    def drain(j, c):
        pltpu.make_async_copy(ones_s_hbm, ones_s, sem_s).wait()
        pltpu.make_async_copy(ones_d_hbm, ones_d, sem_d).wait()
        return c

    lax.fori_loop(0, nb, drain, 0)
    plsc.subcore_barrier()
    pltpu.sync_copy(deg_sh.at[pl.ds(sid * rpt, rpt)], stage)
    pltpu.sync_copy(stage, out_hbm.at[cid, pl.ds(sid * rpt, rpt)])


def _agg_body(np_, h, nb, y_hbm, src_hbm, dst_hbm, zeros_hbm,
              out_hbm, idx_s, idx_d, rows0, rows1, agg_sh,
              sg0, sg1, ss0, ss1):
    cid = lax.axis_index("c")
    sid = lax.axis_index("s")
    wid = sid * NC + cid
    rpt = np_ // NS
    cs = rpt // 8
    stage = rows0.at[pl.ds(0, cs)]
    pltpu.sync_copy(src_hbm.at[wid], idx_s)
    pltpu.sync_copy(dst_hbm.at[wid], idx_d)
    pltpu.sync_copy(zeros_hbm, stage)
    for q in range(8):
        pltpu.sync_copy(stage, agg_sh.at[pl.ds(sid * rpt + q * cs, cs)])
    plsc.subcore_barrier()

    dummy = y_hbm.at[pl.ds(0, K)]

    def gath(j, buf, sem):
        pltpu.async_copy(y_hbm.at[idx_s.at[j]], buf, sem)

    def scat(j, buf, sem):
        pltpu.async_copy(buf, agg_sh.at[idx_d.at[j]], sem, add=True)

    def wait(sem, buf):
        pltpu.make_async_copy(dummy, buf, sem).wait()

    gath(0, rows0, sg0)
    gath(1, rows1, sg1)
    wait(sg0, rows0)
    scat(0, rows0, ss0)
    wait(sg1, rows1)
    scat(1, rows1, ss1)
    wait(ss0, rows0)
    gath(2, rows0, sg0)

    def body(i, c):
        b0 = 2 * i + 2
        wait(ss1, rows1)
        gath(b0 + 1, rows1, sg1)
        wait(sg0, rows0)
        scat(b0, rows0, ss0)
        wait(sg1, rows1)
        scat(b0 + 1, rows1, ss1)
        wait(ss0, rows0)
        gath(b0 + 2, rows0, sg0)
        return c

    lax.fori_loop(0, (nb - 3) // 2, body, 0)
    wait(sg0, rows0)
    scat(nb - 1, rows0, ss0)
    wait(ss1, rows1)
    wait(ss0, rows0)
    plsc.subcore_barrier()
    for q in range(8):
        pltpu.sync_copy(agg_sh.at[pl.ds(sid * rpt + q * cs, cs)], stage)
        pltpu.sync_copy(stage, out_hbm.at[cid, pl.ds(sid * rpt + q * cs, cs)])


def _even_odd_guard(nb):
    if nb % 2 != 1:
        raise ValueError("aggregation pipeline expects an odd block count")


@functools.lru_cache(maxsize=None)
def _make_sc_kernels(np_, h, e):
    nb = e // (NW * K)
    _even_odd_guard(nb)
    nbd = e // (NW * KD)
    mesh = plsc.VectorSubcoreMesh(core_axis_name="c", subcore_axis_name="s")
    params = pltpu.CompilerParams(use_tc_tiling_on_sc=False)
    deg = pl.kernel(
        functools.partial(_deg_body, np_, nbd),
        out_type=jax.ShapeDtypeStruct((NC, np_, 16), jnp.float32),
        mesh=mesh,
        scratch_types=[
            pltpu.VMEM((nbd, KD), jnp.int32),
            pltpu.VMEM((nbd, KD), jnp.int32),
            pltpu.VMEM((KD, 16), jnp.float32),
            pltpu.VMEM((KD, 16), jnp.float32),
            pltpu.VMEM((np_ // NS, 16), jnp.float32),
            pltpu.VMEM_SHARED((np_, 16), jnp.float32),
            pltpu.SemaphoreType.DMA,
            pltpu.SemaphoreType.DMA,
        ],
        compiler_params=params,
    )
    agg = pl.kernel(
        functools.partial(_agg_body, np_, h, nb),
        out_type=jax.ShapeDtypeStruct((NC, np_, h), jnp.float32),
        mesh=mesh,
        scratch_types=[
            pltpu.VMEM((nb, K), jnp.int32),
            pltpu.VMEM((nb, K), jnp.int32),
            pltpu.VMEM((K, h), jnp.float32),
            pltpu.VMEM((K, h), jnp.float32),
            pltpu.VMEM_SHARED((np_, h), jnp.float32),
            pltpu.SemaphoreType.DMA,
            pltpu.SemaphoreType.DMA,
            pltpu.SemaphoreType.DMA,
            pltpu.SemaphoreType.DMA,
        ],
        compiler_params=params,
    )
    return deg, agg


def _norm(col):
    return jnp.where(col > 0, lax.rsqrt(col), 0.0)


def _tc1_body(x_ref, degp_ref, w_ref, y_ref):
    dp = degp_ref[...]
    ns = _norm(dp[0, :, 0:1] + dp[1, :, 0:1])
    y_ref[...] = jnp.dot(x_ref[...], w_ref[...],
                         preferred_element_type=jnp.float32) * ns


def _tc2_body(aggp_ref, degp_ref, b_ref, w_ref, y_ref):
    dp = degp_ref[...]
    nd = _norm(dp[0, :, 1:2] + dp[1, :, 1:2])
    ns = _norm(dp[0, :, 0:1] + dp[1, :, 0:1])
    agg = aggp_ref[0, :, :] + aggp_ref[1, :, :]
    hcur = jnp.maximum(agg * nd + b_ref[...], 0.0)
    y_ref[...] = jnp.dot(hcur, w_ref[...],
                         preferred_element_type=jnp.float32) * ns


def _tc3_body(aggp_ref, degp_ref, b_ref, w_ref, bfc_ref, out_ref):
    dp = degp_ref[...]
    nd = _norm(dp[0, :, 1:2] + dp[1, :, 1:2])
    agg = aggp_ref[0, :, :] + aggp_ref[1, :, :]
    hcur = jnp.maximum(agg * nd + b_ref[...], 0.0)
    out_ref[...] = jnp.dot(hcur, w_ref[...],
                           preferred_element_type=jnp.float32) + bfc_ref[...]


def _row_block(rb, width):
    return pl.BlockSpec((rb, width), lambda i: (i, 0))


def _degp_block(rb):
    return pl.BlockSpec((NC, rb, 16), lambda i: (0, i, 0))


def _full(shape):
    ndim = len(shape)
    return pl.BlockSpec(shape, lambda i: (0,) * ndim)


def kernel(x, edge_index, W1, b1, W2, b2, Wfc, bfc):
    n, d = x.shape
    h = W1.shape[1]
    c = Wfc.shape[1]
    e = edge_index.shape[1]
    rb = 1000
    grid = (n // rb,)

    np_ = ((n + NS * 8 - 1) // (NS * 8)) * (NS * 8)

    src3 = edge_index[0].astype(jnp.int32).reshape(NW, e // (NW * K), K)
    dst3 = edge_index[1].astype(jnp.int32).reshape(NW, e // (NW * K), K)
    srcd = edge_index[0].astype(jnp.int32).reshape(NW, e // (NW * KD), KD)
    dstd = edge_index[1].astype(jnp.int32).reshape(NW, e // (NW * KD), KD)
    ones_s = jnp.zeros((KD, 16), jnp.float32).at[:, 0].set(1.0)
    ones_d = jnp.zeros((KD, 16), jnp.float32).at[:, 1].set(1.0)
    zeros8 = jnp.zeros((np_ // NS, 16), jnp.float32)
    zerosh = jnp.zeros((np_ // NS // 8, h), jnp.float32)

    deg_k, agg_k = _make_sc_kernels(np_, h, e)
    degp = deg_k(srcd, dstd, ones_s, ones_d, zeros8)

    tc1 = pl.pallas_call(
        _tc1_body,
        grid=grid,
        in_specs=[_row_block(rb, d), _degp_block(rb), _full((d, h))],
        out_specs=_row_block(rb, h),
        out_shape=jax.ShapeDtypeStruct((n, h), jnp.float32),
    )
    y1 = tc1(x, degp, W1)
    aggp1 = agg_k(y1, src3, dst3, zerosh)

    tc2 = pl.pallas_call(
        _tc2_body,
        grid=grid,
        in_specs=[pl.BlockSpec((NC, rb, h), lambda i: (0, i, 0)),
                  _degp_block(rb), _full((1, h)), _full((h, h))],
        out_specs=_row_block(rb, h),
        out_shape=jax.ShapeDtypeStruct((n, h), jnp.float32),
    )
    y2 = tc2(aggp1, degp, b1.reshape(1, h), W2)
    aggp2 = agg_k(y2, src3, dst3, zerosh)

    tc3 = pl.pallas_call(
        _tc3_body,
        grid=grid,
        in_specs=[pl.BlockSpec((NC, rb, h), lambda i: (0, i, 0)),
                  _degp_block(rb), _full((1, h)), _full((h, c)),
                  _full((1, c))],
        out_specs=_row_block(rb, c),
        out_shape=jax.ShapeDtypeStruct((n, c), jnp.float32),
    )
    return tc3(aggp2, degp, b2.reshape(1, h), Wfc, bfc.reshape(1, c))


# speedup vs baseline: 1.2007x; 1.2007x over previous
import functools

import jax
import jax.numpy as jnp
from jax import lax
from jax.experimental import pallas as pl
from jax.experimental.pallas import tpu as pltpu
from jax.experimental.pallas import tpu_sc as plsc

NC = 2
NS = 16
NW = NC * NS
K = 80
KD = 400


def _deg_body(np_, nb, src_hbm, dst_hbm, ones_s_hbm, ones_d_hbm, zeros_hbm,
              out_hbm, idx_s, idx_d, ones_s, ones_d, stage, deg_sh,
              sem_s, sem_d):
    cid = lax.axis_index("c")
    sid = lax.axis_index("s")
    wid = sid * NC + cid
    rpt = np_ // NS
    pltpu.sync_copy(src_hbm.at[wid], idx_s)
    pltpu.sync_copy(dst_hbm.at[wid], idx_d)
    pltpu.sync_copy(ones_s_hbm, ones_s)
    pltpu.sync_copy(ones_d_hbm, ones_d)
    pltpu.sync_copy(zeros_hbm, stage)
    pltpu.sync_copy(stage, deg_sh.at[pl.ds(sid * rpt, rpt)])
    plsc.subcore_barrier()

    def body(j, c):
        pltpu.async_copy(ones_s, deg_sh.at[idx_s.at[j]], sem_s, add=True)
        pltpu.async_copy(ones_d, deg_sh.at[idx_d.at[j]], sem_d, add=True)
        return c

    lax.fori_loop(0, nb, body, 0)

    def drain(j, c):
        pltpu.make_async_copy(ones_s_hbm, ones_s, sem_s).wait()
        pltpu.make_async_copy(ones_d_hbm, ones_d, sem_d).wait()
        return c

    lax.fori_loop(0, nb, drain, 0)
    plsc.subcore_barrier()
    pltpu.sync_copy(deg_sh.at[pl.ds(sid * rpt, rpt)], stage)
    pltpu.sync_copy(stage, out_hbm.at[cid, pl.ds(sid * rpt, rpt)])


def _agg_body(np_, h, nb, y_hbm, src_hbm, dst_hbm, zeros_hbm,
              out_hbm, idx_s, idx_d, rows0, rows1, agg_sh, sg0, sg1):
    cid = lax.axis_index("c")
    sid = lax.axis_index("s")
    wid = sid * NC + cid
    rpt = np_ // NS
    cs = rpt // 8
    stage = rows0.at[pl.ds(0, cs)]
    pltpu.sync_copy(src_hbm.at[wid], idx_s)
    pltpu.sync_copy(dst_hbm.at[wid], idx_d)
    pltpu.sync_copy(zeros_hbm, stage)
    for q in range(8):
        pltpu.sync_copy(stage, agg_sh.at[pl.ds(sid * rpt + q * cs, cs)])
    plsc.subcore_barrier()

    dummy = y_hbm.at[pl.ds(0, K)]
    pltpu.async_copy(y_hbm.at[idx_s.at[0]], rows0, sg0)

    def body(i, c):
        b0 = 2 * i
        pltpu.async_copy(y_hbm.at[idx_s.at[b0 + 1]], rows1, sg1)
        pltpu.make_async_copy(dummy, rows0, sg0).wait()
        pltpu.sync_copy(rows0, agg_sh.at[idx_d.at[b0]], add=True)
        pltpu.async_copy(y_hbm.at[idx_s.at[b0 + 2]], rows0, sg0)
        pltpu.make_async_copy(dummy, rows1, sg1).wait()
        pltpu.sync_copy(rows1, agg_sh.at[idx_d.at[b0 + 1]], add=True)
        return c

    lax.fori_loop(0, (nb - 1) // 2, body, 0)
    pltpu.make_async_copy(dummy, rows0, sg0).wait()
    pltpu.sync_copy(rows0, agg_sh.at[idx_d.at[nb - 1]], add=True)
    plsc.subcore_barrier()
    for q in range(8):
        pltpu.sync_copy(agg_sh.at[pl.ds(sid * rpt + q * cs, cs)], stage)
        pltpu.sync_copy(stage, out_hbm.at[cid, pl.ds(sid * rpt + q * cs, cs)])


def _even_odd_guard(nb):
    if nb % 2 != 1:
        raise ValueError("aggregation pipeline expects an odd block count")


@functools.lru_cache(maxsize=None)
def _make_sc_kernels(np_, h, e):
    nb = e // (NW * K)
    _even_odd_guard(nb)
    nbd = e // (NW * KD)
    mesh = plsc.VectorSubcoreMesh(core_axis_name="c", subcore_axis_name="s")
    params = pltpu.CompilerParams(use_tc_tiling_on_sc=False)
    deg = pl.kernel(
        functools.partial(_deg_body, np_, nbd),
        out_type=jax.ShapeDtypeStruct((NC, np_, 16), jnp.float32),
        mesh=mesh,
        scratch_types=[
            pltpu.VMEM((nbd, KD), jnp.int32),
            pltpu.VMEM((nbd, KD), jnp.int32),
            pltpu.VMEM((KD, 16), jnp.float32),
            pltpu.VMEM((KD, 16), jnp.float32),
            pltpu.VMEM((np_ // NS, 16), jnp.float32),
            pltpu.VMEM_SHARED((np_, 16), jnp.float32),
            pltpu.SemaphoreType.DMA,
            pltpu.SemaphoreType.DMA,
        ],
        compiler_params=params,
    )
    agg = pl.kernel(
        functools.partial(_agg_body, np_, h, nb),
        out_type=jax.ShapeDtypeStruct((NC, np_, h), jnp.float32),
        mesh=mesh,
        scratch_types=[
            pltpu.VMEM((nb, K), jnp.int32),
            pltpu.VMEM((nb, K), jnp.int32),
            pltpu.VMEM((K, h), jnp.float32),
            pltpu.VMEM((K, h), jnp.float32),
            pltpu.VMEM_SHARED((np_, h), jnp.float32),
            pltpu.SemaphoreType.DMA,
            pltpu.SemaphoreType.DMA,
        ],
        compiler_params=params,
    )
    return deg, agg


def _norm(col):
    return jnp.where(col > 0, lax.rsqrt(col), 0.0)


def _tc1_body(x_ref, degp_ref, w_ref, y_ref):
    dp = degp_ref[...]
    ns = _norm(dp[0, :, 0:1] + dp[1, :, 0:1])
    y_ref[...] = jnp.dot(x_ref[...], w_ref[...],
                         preferred_element_type=jnp.float32) * ns


def _tc2_body(aggp_ref, degp_ref, b_ref, w_ref, y_ref):
    dp = degp_ref[...]
    nd = _norm(dp[0, :, 1:2] + dp[1, :, 1:2])
    ns = _norm(dp[0, :, 0:1] + dp[1, :, 0:1])
    agg = aggp_ref[0, :, :] + aggp_ref[1, :, :]
    hcur = jnp.maximum(agg * nd + b_ref[...], 0.0)
    y_ref[...] = jnp.dot(hcur, w_ref[...],
                         preferred_element_type=jnp.float32) * ns


def _tc3_body(aggp_ref, degp_ref, b_ref, w_ref, bfc_ref, out_ref):
    dp = degp_ref[...]
    nd = _norm(dp[0, :, 1:2] + dp[1, :, 1:2])
    agg = aggp_ref[0, :, :] + aggp_ref[1, :, :]
    hcur = jnp.maximum(agg * nd + b_ref[...], 0.0)
    out_ref[...] = jnp.dot(hcur, w_ref[...],
                           preferred_element_type=jnp.float32) + bfc_ref[...]


def _row_block(rb, width):
    return pl.BlockSpec((rb, width), lambda i: (i, 0))


def _degp_block(rb):
    return pl.BlockSpec((NC, rb, 16), lambda i: (0, i, 0))


def _full(shape):
    ndim = len(shape)
    return pl.BlockSpec(shape, lambda i: (0,) * ndim)


def kernel(x, edge_index, W1, b1, W2, b2, Wfc, bfc):
    n, d = x.shape
    h = W1.shape[1]
    c = Wfc.shape[1]
    e = edge_index.shape[1]
    rb = 1000
    grid = (n // rb,)

    np_ = ((n + NS * 8 - 1) // (NS * 8)) * (NS * 8)

    src3 = edge_index[0].astype(jnp.int32).reshape(NW, e // (NW * K), K)
    dst3 = edge_index[1].astype(jnp.int32).reshape(NW, e // (NW * K), K)
    srcd = edge_index[0].astype(jnp.int32).reshape(NW, e // (NW * KD), KD)
    dstd = edge_index[1].astype(jnp.int32).reshape(NW, e // (NW * KD), KD)
    ones_s = jnp.zeros((KD, 16), jnp.float32).at[:, 0].set(1.0)
    ones_d = jnp.zeros((KD, 16), jnp.float32).at[:, 1].set(1.0)
    zeros8 = jnp.zeros((np_ // NS, 16), jnp.float32)
    zerosh = jnp.zeros((np_ // NS // 8, h), jnp.float32)

    deg_k, agg_k = _make_sc_kernels(np_, h, e)
    degp = deg_k(srcd, dstd, ones_s, ones_d, zeros8)

    tc1 = pl.pallas_call(
        _tc1_body,
        grid=grid,
        in_specs=[_row_block(rb, d), _degp_block(rb), _full((d, h))],
        out_specs=_row_block(rb, h),
        out_shape=jax.ShapeDtypeStruct((n, h), jnp.float32),
    )
    y1 = tc1(x, degp, W1)
    aggp1 = agg_k(y1, src3, dst3, zerosh)

    tc2 = pl.pallas_call(
        _tc2_body,
        grid=grid,
        in_specs=[pl.BlockSpec((NC, rb, h), lambda i: (0, i, 0)),
                  _degp_block(rb), _full((1, h)), _full((h, h))],
        out_specs=_row_block(rb, h),
        out_shape=jax.ShapeDtypeStruct((n, h), jnp.float32),
    )
    y2 = tc2(aggp1, degp, b1.reshape(1, h), W2)
    aggp2 = agg_k(y2, src3, dst3, zerosh)

    tc3 = pl.pallas_call(
        _tc3_body,
        grid=grid,
        in_specs=[pl.BlockSpec((NC, rb, h), lambda i: (0, i, 0)),
                  _degp_block(rb), _full((1, h)), _full((h, c)),
                  _full((1, c))],
        out_specs=_row_block(rb, c),
        out_shape=jax.ShapeDtypeStruct((n, c), jnp.float32),
    )
    return tc3(aggp2, degp, b2.reshape(1, h), Wfc, bfc.reshape(1, c))


# speedup vs baseline: 1.2428x; 1.0350x over previous
import functools

import jax
import jax.numpy as jnp
from jax import lax
from jax.experimental import pallas as pl
from jax.experimental.pallas import tpu as pltpu
from jax.experimental.pallas import tpu_sc as plsc

NC = 2
NS = 16
NW = NC * NS
K = 80


def _deg_body(np_, nb, edges_hbm, ones_s_hbm, ones_d_hbm, zeros_hbm,
              out_hbm, idx_s, idx_d, ones_s, ones_d, stage, deg_sh,
              sem_s, sem_d):
    cid = lax.axis_index("c")
    sid = lax.axis_index("s")
    wid = sid * NC + cid
    rpt = np_ // NS
    pltpu.sync_copy(edges_hbm.at[0, wid], idx_s)
    pltpu.sync_copy(edges_hbm.at[1, wid], idx_d)
    pltpu.sync_copy(ones_s_hbm, ones_s)
    pltpu.sync_copy(ones_d_hbm, ones_d)
    pltpu.sync_copy(zeros_hbm, stage)
    pltpu.sync_copy(stage, deg_sh.at[pl.ds(sid * rpt, rpt)])
    plsc.subcore_barrier()

    def body(j, c):
        pltpu.async_copy(ones_s, deg_sh.at[idx_s.at[j]], sem_s, add=True)
        pltpu.async_copy(ones_d, deg_sh.at[idx_d.at[j]], sem_d, add=True)
        return c

    lax.fori_loop(0, nb, body, 0)

    def drain(j, c):
        pltpu.make_async_copy(ones_s_hbm, ones_s, sem_s).wait()
        pltpu.make_async_copy(ones_d_hbm, ones_d, sem_d).wait()
        return c

    lax.fori_loop(0, nb, drain, 0)
    plsc.subcore_barrier()
    pltpu.sync_copy(deg_sh.at[pl.ds(sid * rpt, rpt)], stage)
    pltpu.sync_copy(stage, out_hbm.at[cid, pl.ds(sid * rpt, rpt)])


def _agg_body(np_, h, nb, y_hbm, edges_hbm, zeros_hbm,
              out_hbm, idx_s, idx_d, rows0, rows1, agg_sh, sg0, sg1):
    cid = lax.axis_index("c")
    sid = lax.axis_index("s")
    wid = sid * NC + cid
    rpt = np_ // NS
    cs = rpt // 8
    stage = rows0.at[pl.ds(0, cs)]
    pltpu.sync_copy(edges_hbm.at[0, wid], idx_s)
    pltpu.sync_copy(edges_hbm.at[1, wid], idx_d)
    pltpu.sync_copy(zeros_hbm, stage)
    for q in range(8):
        pltpu.sync_copy(stage, agg_sh.at[pl.ds(sid * rpt + q * cs, cs)])
    plsc.subcore_barrier()

    dummy = y_hbm.at[pl.ds(0, K)]
    pltpu.async_copy(y_hbm.at[idx_s.at[0]], rows0, sg0)

    def body(i, c):
        b0 = 2 * i
        pltpu.async_copy(y_hbm.at[idx_s.at[b0 + 1]], rows1, sg1)
        pltpu.make_async_copy(dummy, rows0, sg0).wait()
        pltpu.sync_copy(rows0, agg_sh.at[idx_d.at[b0]], add=True)
        pltpu.async_copy(y_hbm.at[idx_s.at[b0 + 2]], rows0, sg0)
        pltpu.make_async_copy(dummy, rows1, sg1).wait()
        pltpu.sync_copy(rows1, agg_sh.at[idx_d.at[b0 + 1]], add=True)
        return c

    lax.fori_loop(0, (nb - 1) // 2, body, 0)
    pltpu.make_async_copy(dummy, rows0, sg0).wait()
    pltpu.sync_copy(rows0, agg_sh.at[idx_d.at[nb - 1]], add=True)
    plsc.subcore_barrier()
    for q in range(8):
        pltpu.sync_copy(agg_sh.at[pl.ds(sid * rpt + q * cs, cs)], stage)
        pltpu.sync_copy(stage, out_hbm.at[cid, pl.ds(sid * rpt + q * cs, cs)])


def _even_odd_guard(nb):
    if nb % 2 != 1:
        raise ValueError("aggregation pipeline expects an odd block count")


@functools.lru_cache(maxsize=None)
def _make_sc_kernels(np_, h, e):
    nb = e // (NW * K)
    _even_odd_guard(nb)
    mesh = plsc.VectorSubcoreMesh(core_axis_name="c", subcore_axis_name="s")
    params = pltpu.CompilerParams(use_tc_tiling_on_sc=False)
    deg = pl.kernel(
        functools.partial(_deg_body, np_, nb),
        out_type=jax.ShapeDtypeStruct((NC, np_, 16), jnp.float32),
        mesh=mesh,
        scratch_types=[
            pltpu.VMEM((nb, K), jnp.int32),
            pltpu.VMEM((nb, K), jnp.int32),
            pltpu.VMEM((K, 16), jnp.float32),
            pltpu.VMEM((K, 16), jnp.float32),
            pltpu.VMEM((np_ // NS, 16), jnp.float32),
            pltpu.VMEM_SHARED((np_, 16), jnp.float32),
            pltpu.SemaphoreType.DMA,
            pltpu.SemaphoreType.DMA,
        ],
        compiler_params=params,
    )
    agg = pl.kernel(
        functools.partial(_agg_body, np_, h, nb),
        out_type=jax.ShapeDtypeStruct((NC, np_, h), jnp.float32),
        mesh=mesh,
        scratch_types=[
            pltpu.VMEM((nb, K), jnp.int32),
            pltpu.VMEM((nb, K), jnp.int32),
            pltpu.VMEM((K, h), jnp.float32),
            pltpu.VMEM((K, h), jnp.float32),
            pltpu.VMEM_SHARED((np_, h), jnp.float32),
            pltpu.SemaphoreType.DMA,
            pltpu.SemaphoreType.DMA,
        ],
        compiler_params=params,
    )
    return deg, agg


def _norm(col):
    return jnp.where(col > 0, lax.rsqrt(col), 0.0)


def _tc1_body(x_ref, degp_ref, w_ref, y_ref):
    dp = degp_ref[...]
    ns = _norm(dp[0, :, 0:1] + dp[1, :, 0:1])
    y_ref[...] = jnp.dot(x_ref[...], w_ref[...],
                         preferred_element_type=jnp.float32) * ns


def _tc2_body(aggp_ref, degp_ref, b_ref, w_ref, y_ref):
    dp = degp_ref[...]
    nd = _norm(dp[0, :, 1:2] + dp[1, :, 1:2])
    ns = _norm(dp[0, :, 0:1] + dp[1, :, 0:1])
    agg = aggp_ref[0, :, :] + aggp_ref[1, :, :]
    hcur = jnp.maximum(agg * nd + b_ref[...], 0.0)
    y_ref[...] = jnp.dot(hcur, w_ref[...],
                         preferred_element_type=jnp.float32) * ns


def _tc3_body(aggp_ref, degp_ref, b_ref, w_ref, bfc_ref, out_ref):
    dp = degp_ref[...]
    nd = _norm(dp[0, :, 1:2] + dp[1, :, 1:2])
    agg = aggp_ref[0, :, :] + aggp_ref[1, :, :]
    hcur = jnp.maximum(agg * nd + b_ref[...], 0.0)
    out_ref[...] = jnp.dot(hcur, w_ref[...],
                           preferred_element_type=jnp.float32) + bfc_ref[...]


def _row_block(rb, width):
    return pl.BlockSpec((rb, width), lambda i: (i, 0))


def _degp_block(rb):
    return pl.BlockSpec((NC, rb, 16), lambda i: (0, i, 0))


def _full(shape):
    ndim = len(shape)
    return pl.BlockSpec(shape, lambda i: (0,) * ndim)


def kernel(x, edge_index, W1, b1, W2, b2, Wfc, bfc):
    n, d = x.shape
    h = W1.shape[1]
    c = Wfc.shape[1]
    e = edge_index.shape[1]
    rb = 1000
    grid = (n // rb,)

    np_ = ((n + NS * 8 - 1) // (NS * 8)) * (NS * 8)

    e3 = edge_index.astype(jnp.int32).reshape(2, NW, e // (NW * K), K)
    ones_s = jnp.zeros((K, 16), jnp.float32).at[:, 0].set(1.0)
    ones_d = jnp.zeros((K, 16), jnp.float32).at[:, 1].set(1.0)
    zeros8 = jnp.zeros((np_ // NS, 16), jnp.float32)
    zerosh = jnp.zeros((np_ // NS // 8, h), jnp.float32)

    deg_k, agg_k = _make_sc_kernels(np_, h, e)
    degp = deg_k(e3, ones_s, ones_d, zeros8)

    tc1 = pl.pallas_call(
        _tc1_body,
        grid=grid,
        in_specs=[_row_block(rb, d), _degp_block(rb), _full((d, h))],
        out_specs=_row_block(rb, h),
        out_shape=jax.ShapeDtypeStruct((n, h), jnp.float32),
    )
    y1 = tc1(x, degp, W1)
    aggp1 = agg_k(y1, e3, zerosh)

    tc2 = pl.pallas_call(
        _tc2_body,
        grid=grid,
        in_specs=[pl.BlockSpec((NC, rb, h), lambda i: (0, i, 0)),
                  _degp_block(rb), _full((1, h)), _full((h, h))],
        out_specs=_row_block(rb, h),
        out_shape=jax.ShapeDtypeStruct((n, h), jnp.float32),
    )
    y2 = tc2(aggp1, degp, b1.reshape(1, h), W2)
    aggp2 = agg_k(y2, e3, zerosh)

    tc3 = pl.pallas_call(
        _tc3_body,
        grid=grid,
        in_specs=[pl.BlockSpec((NC, rb, h), lambda i: (0, i, 0)),
                  _degp_block(rb), _full((1, h)), _full((h, c)),
                  _full((1, c))],
        out_specs=_row_block(rb, c),
        out_shape=jax.ShapeDtypeStruct((n, c), jnp.float32),
    )
    return tc3(aggp2, degp, b2.reshape(1, h), Wfc, bfc.reshape(1, c))


# speedup vs baseline: 1.2529x; 1.0082x over previous
import functools

import jax
import jax.numpy as jnp
import numpy as np
from jax import lax
from jax.experimental import pallas as pl
from jax.experimental.pallas import tpu as pltpu
from jax.experimental.pallas import tpu_sc as plsc

NC = 2
NS = 16
NW = NC * NS
K = 80


def _deg_body(np_, nb, edges_hbm, ones_s_hbm, ones_d_hbm, zeros_hbm,
              out_hbm, idx_s, idx_d, ones_s, ones_d, stage, deg_sh,
              sem_s, sem_d):
    cid = lax.axis_index("c")
    sid = lax.axis_index("s")
    wid = sid * NC + cid
    rpt = np_ // NS
    pltpu.sync_copy(edges_hbm.at[0, wid], idx_s)
    pltpu.sync_copy(edges_hbm.at[1, wid], idx_d)
    pltpu.sync_copy(ones_s_hbm, ones_s)
    pltpu.sync_copy(ones_d_hbm, ones_d)
    pltpu.sync_copy(zeros_hbm, stage)
    pltpu.sync_copy(stage, deg_sh.at[pl.ds(sid * rpt, rpt)])
    plsc.subcore_barrier()

    def body(j, c):
        pltpu.async_copy(ones_s, deg_sh.at[idx_s.at[j]], sem_s, add=True)
        pltpu.async_copy(ones_d, deg_sh.at[idx_d.at[j]], sem_d, add=True)
        return c

    lax.fori_loop(0, nb, body, 0)

    def drain(j, c):
        pltpu.make_async_copy(ones_s_hbm, ones_s, sem_s).wait()
        pltpu.make_async_copy(ones_d_hbm, ones_d, sem_d).wait()
        return c

    lax.fori_loop(0, nb, drain, 0)
    plsc.subcore_barrier()
    pltpu.sync_copy(deg_sh.at[pl.ds(sid * rpt, rpt)], stage)
    pltpu.sync_copy(stage, out_hbm.at[cid, pl.ds(sid * rpt, rpt)])


def _agg_body(np_, h, nb, y_hbm, edges_hbm, zeros_hbm,
              out_hbm, idx_s, idx_d, rows0, rows1, agg_sh, sg0, sg1):
    cid = lax.axis_index("c")
    sid = lax.axis_index("s")
    wid = sid * NC + cid
    rpt = np_ // NS
    cs = rpt // 8
    stage = rows0.at[pl.ds(0, cs)]
    pltpu.sync_copy(edges_hbm.at[0, wid], idx_s)
    pltpu.sync_copy(edges_hbm.at[1, wid], idx_d)
    pltpu.sync_copy(zeros_hbm, stage)
    for q in range(8):
        pltpu.sync_copy(stage, agg_sh.at[pl.ds(sid * rpt + q * cs, cs)])
    plsc.subcore_barrier()

    dummy = y_hbm.at[pl.ds(0, K)]
    pltpu.async_copy(y_hbm.at[idx_s.at[0]], rows0, sg0)

    def body(i, c):
        b0 = 2 * i
        pltpu.async_copy(y_hbm.at[idx_s.at[b0 + 1]], rows1, sg1)
        pltpu.make_async_copy(dummy, rows0, sg0).wait()
        pltpu.sync_copy(rows0, agg_sh.at[idx_d.at[b0]], add=True)
        pltpu.async_copy(y_hbm.at[idx_s.at[b0 + 2]], rows0, sg0)
        pltpu.make_async_copy(dummy, rows1, sg1).wait()
        pltpu.sync_copy(rows1, agg_sh.at[idx_d.at[b0 + 1]], add=True)
        return c

    lax.fori_loop(0, (nb - 1) // 2, body, 0)
    pltpu.make_async_copy(dummy, rows0, sg0).wait()
    pltpu.sync_copy(rows0, agg_sh.at[idx_d.at[nb - 1]], add=True)
    plsc.subcore_barrier()
    for q in range(8):
        pltpu.sync_copy(agg_sh.at[pl.ds(sid * rpt + q * cs, cs)], stage)
        pltpu.sync_copy(stage, out_hbm.at[cid, pl.ds(sid * rpt + q * cs, cs)])


def _even_odd_guard(nb):
    if nb % 2 != 1:
        raise ValueError("aggregation pipeline expects an odd block count")


@functools.lru_cache(maxsize=None)
def _make_sc_kernels(np_, h, e):
    nb = e // (NW * K)
    _even_odd_guard(nb)
    mesh = plsc.VectorSubcoreMesh(core_axis_name="c", subcore_axis_name="s")
    params = pltpu.CompilerParams(use_tc_tiling_on_sc=False)
    deg = pl.kernel(
        functools.partial(_deg_body, np_, nb),
        out_type=jax.ShapeDtypeStruct((NC, np_, 16), jnp.float32),
        mesh=mesh,
        scratch_types=[
            pltpu.VMEM((nb, K), jnp.int32),
            pltpu.VMEM((nb, K), jnp.int32),
            pltpu.VMEM((K, 16), jnp.float32),
            pltpu.VMEM((K, 16), jnp.float32),
            pltpu.VMEM((np_ // NS, 16), jnp.float32),
            pltpu.VMEM_SHARED((np_, 16), jnp.float32),
            pltpu.SemaphoreType.DMA,
            pltpu.SemaphoreType.DMA,
        ],
        compiler_params=params,
    )
    agg = pl.kernel(
        functools.partial(_agg_body, np_, h, nb),
        out_type=jax.ShapeDtypeStruct((NC, np_, h), jnp.float32),
        mesh=mesh,
        scratch_types=[
            pltpu.VMEM((nb, K), jnp.int32),
            pltpu.VMEM((nb, K), jnp.int32),
            pltpu.VMEM((K, h), jnp.float32),
            pltpu.VMEM((K, h), jnp.float32),
            pltpu.VMEM_SHARED((np_, h), jnp.float32),
            pltpu.SemaphoreType.DMA,
            pltpu.SemaphoreType.DMA,
        ],
        compiler_params=params,
    )
    return deg, agg


def _norm(col):
    return jnp.where(col > 0, lax.rsqrt(col), 0.0)


def _tc1_body(x_ref, degp_ref, w_ref, y_ref):
    dp = degp_ref[...]
    ns = _norm(dp[0, :, 0:1] + dp[1, :, 0:1])
    y_ref[...] = jnp.dot(x_ref[...], w_ref[...],
                         preferred_element_type=jnp.float32) * ns


def _tc2_body(aggp_ref, degp_ref, b_ref, w_ref, y_ref):
    dp = degp_ref[...]
    nd = _norm(dp[0, :, 1:2] + dp[1, :, 1:2])
    ns = _norm(dp[0, :, 0:1] + dp[1, :, 0:1])
    agg = aggp_ref[0, :, :] + aggp_ref[1, :, :]
    hcur = jnp.maximum(agg * nd + b_ref[...], 0.0)
    y_ref[...] = jnp.dot(hcur, w_ref[...],
                         preferred_element_type=jnp.float32) * ns


def _tc3_body(aggp_ref, degp_ref, b_ref, w_ref, bfc_ref, out_ref):
    dp = degp_ref[...]
    nd = _norm(dp[0, :, 1:2] + dp[1, :, 1:2])
    agg = aggp_ref[0, :, :] + aggp_ref[1, :, :]
    hcur = jnp.maximum(agg * nd + b_ref[...], 0.0)
    out_ref[...] = jnp.dot(hcur, w_ref[...],
                           preferred_element_type=jnp.float32) + bfc_ref[...]


def _row_block(rb, width):
    return pl.BlockSpec((rb, width), lambda i: (i, 0))


def _degp_block(rb):
    return pl.BlockSpec((NC, rb, 16), lambda i: (0, i, 0))


def _full(shape):
    ndim = len(shape)
    return pl.BlockSpec(shape, lambda i: (0,) * ndim)


def kernel(x, edge_index, W1, b1, W2, b2, Wfc, bfc):
    n, d = x.shape
    h = W1.shape[1]
    c = Wfc.shape[1]
    e = edge_index.shape[1]
    rb = 1000
    grid = (n // rb,)

    np_ = ((n + NS * 8 - 1) // (NS * 8)) * (NS * 8)

    e3 = edge_index.astype(jnp.int32).reshape(2, NW, e // (NW * K), K)
    ones_np_s = np.zeros((K, 16), np.float32)
    ones_np_s[:, 0] = 1.0
    ones_np_d = np.zeros((K, 16), np.float32)
    ones_np_d[:, 1] = 1.0
    ones_s = jnp.asarray(ones_np_s)
    ones_d = jnp.asarray(ones_np_d)
    zeros8 = jnp.asarray(np.zeros((np_ // NS, 16), np.float32))
    zerosh = jnp.asarray(np.zeros((np_ // NS // 8, h), np.float32))

    deg_k, agg_k = _make_sc_kernels(np_, h, e)
    degp = deg_k(e3, ones_s, ones_d, zeros8)

    tc1 = pl.pallas_call(
        _tc1_body,
        grid=grid,
        in_specs=[_row_block(rb, d), _degp_block(rb), _full((d, h))],
        out_specs=_row_block(rb, h),
        out_shape=jax.ShapeDtypeStruct((n, h), jnp.float32),
    )
    y1 = tc1(x, degp, W1)
    aggp1 = agg_k(y1, e3, zerosh)

    tc2 = pl.pallas_call(
        _tc2_body,
        grid=grid,
        in_specs=[pl.BlockSpec((NC, rb, h), lambda i: (0, i, 0)),
                  _degp_block(rb), _full((1, h)), _full((h, h))],
        out_specs=_row_block(rb, h),
        out_shape=jax.ShapeDtypeStruct((n, h), jnp.float32),
    )
    y2 = tc2(aggp1, degp, b1.reshape(1, h), W2)
    aggp2 = agg_k(y2, e3, zerosh)

    tc3 = pl.pallas_call(
        _tc3_body,
        grid=grid,
        in_specs=[pl.BlockSpec((NC, rb, h), lambda i: (0, i, 0)),
                  _degp_block(rb), _full((1, h)), _full((h, c)),
                  _full((1, c))],
        out_specs=_row_block(rb, c),
        out_shape=jax.ShapeDtypeStruct((n, c), jnp.float32),
    )
    return tc3(aggp2, degp, b2.reshape(1, h), Wfc, bfc.reshape(1, c))


# speedup vs baseline: 1.4450x; 1.1533x over previous
import functools

import jax
import jax.numpy as jnp
import numpy as np
from jax import lax
from jax.experimental import pallas as pl
from jax.experimental.pallas import tpu as pltpu
from jax.experimental.pallas import tpu_sc as plsc

NC = 2
NS = 16
NW = NC * NS
K = 80


def _deg_body(np_, nb, edges_hbm, ones_s_hbm, ones_d_hbm, zeros_hbm,
              out_hbm, idx_s, idx_d, ones_s, ones_d, stage, deg_sh,
              sem_s, sem_d):
    cid = lax.axis_index("c")
    sid = lax.axis_index("s")
    wid = sid * NC + cid
    rpt = np_ // NS
    pltpu.sync_copy(edges_hbm.at[0, wid], idx_s)
    pltpu.sync_copy(edges_hbm.at[1, wid], idx_d)
    pltpu.sync_copy(ones_s_hbm, ones_s)
    pltpu.sync_copy(ones_d_hbm, ones_d)
    pltpu.sync_copy(zeros_hbm, stage)
    pltpu.sync_copy(stage, deg_sh.at[pl.ds(sid * rpt, rpt)])
    plsc.subcore_barrier()

    def body(j, c):
        pltpu.async_copy(ones_s, deg_sh.at[idx_s.at[j]], sem_s, add=True)
        pltpu.async_copy(ones_d, deg_sh.at[idx_d.at[j]], sem_d, add=True)
        return c

    lax.fori_loop(0, nb, body, 0)

    def drain(j, c):
        pltpu.make_async_copy(ones_s_hbm, ones_s, sem_s).wait()
        pltpu.make_async_copy(ones_d_hbm, ones_d, sem_d).wait()
        return c

    lax.fori_loop(0, nb, drain, 0)
    plsc.subcore_barrier()
    pltpu.sync_copy(deg_sh.at[pl.ds(sid * rpt, rpt)], stage)
    pltpu.sync_copy(stage, out_hbm.at[cid, pl.ds(sid * rpt, rpt)])


def _agg_body(nn, h, nb, y_hbm, edges_hbm, zeros_hbm,
              out_hbm, idx_s, idx_d, rows0, rows1, rows2, agg_sh,
              sg0, sg1, sg2):
    cid = lax.axis_index("c")
    sid = lax.axis_index("s")
    wid = sid * NC + cid
    rpt = ((nn + NS - 1) // NS + 7) // 8 * 8
    cs = rpt // 8
    last = nn - rpt * (NS - 1)
    csl = 40
    if last <= 0 or last % csl or csl > cs or last // csl > 16:
        raise ValueError("bad last-tile chunking")
    stage = rows0.at[pl.ds(0, cs)]
    stagel = rows0.at[pl.ds(0, csl)]
    pltpu.sync_copy(edges_hbm.at[0, wid], idx_s)
    pltpu.sync_copy(edges_hbm.at[1, wid], idx_d)

    @pl.when(sid < NS - 1)
    def _():
        pltpu.sync_copy(zeros_hbm.at[pl.ds(0, cs)], stage)
        for q in range(8):
            pltpu.sync_copy(stage, agg_sh.at[pl.ds(sid * rpt + q * cs, cs)])

    @pl.when(sid == NS - 1)
    def _():
        pltpu.sync_copy(zeros_hbm.at[pl.ds(0, csl)], stagel)
        for q in range(last // csl):
            pltpu.sync_copy(
                stagel, agg_sh.at[pl.ds((NS - 1) * rpt + q * csl, csl)])

    plsc.subcore_barrier()

    dummy = y_hbm.at[pl.ds(0, K)]

    def gath(j, buf, sem):
        pltpu.async_copy(y_hbm.at[idx_s.at[j]], buf, sem)

    def wait(sem, buf):
        pltpu.make_async_copy(dummy, buf, sem).wait()

    def scat(j, buf):
        pltpu.sync_copy(buf, agg_sh.at[idx_d.at[j]], add=True)

    gath(0, rows0, sg0)
    gath(1, rows1, sg1)

    def body(i, c):
        b = 3 * i
        gath(b + 2, rows2, sg2)
        wait(sg0, rows0)
        scat(b, rows0)
        gath(b + 3, rows0, sg0)
        wait(sg1, rows1)
        scat(b + 1, rows1)
        gath(b + 4, rows1, sg1)
        wait(sg2, rows2)
        scat(b + 2, rows2)
        return c

    lax.fori_loop(0, (nb - 2) // 3, body, 0)
    wait(sg0, rows0)
    scat(nb - 2, rows0)
    wait(sg1, rows1)
    scat(nb - 1, rows1)
    plsc.subcore_barrier()

    @pl.when(sid < NS - 1)
    def _():
        for q in range(8):
            pltpu.sync_copy(agg_sh.at[pl.ds(sid * rpt + q * cs, cs)], stage)
            pltpu.sync_copy(stage,
                            out_hbm.at[cid, pl.ds(sid * rpt + q * cs, cs)])

    @pl.when(sid == NS - 1)
    def _():
        for q in range(last // csl):
            o = (NS - 1) * rpt + q * csl
            pltpu.sync_copy(agg_sh.at[pl.ds(o, csl)], stagel)
            pltpu.sync_copy(stagel, out_hbm.at[cid, pl.ds(o, csl)])


def _even_odd_guard(nb):
    if nb < 2 or nb % 3 != 2:
        raise ValueError("aggregation pipeline expects nb % 3 == 2")


@functools.lru_cache(maxsize=None)
def _make_sc_kernels(np_, nn, h, e):
    nb = e // (NW * K)
    _even_odd_guard(nb)
    mesh = plsc.VectorSubcoreMesh(core_axis_name="c", subcore_axis_name="s")
    params = pltpu.CompilerParams(use_tc_tiling_on_sc=False)
    deg = pl.kernel(
        functools.partial(_deg_body, np_, nb),
        out_type=jax.ShapeDtypeStruct((NC, np_, 16), jnp.float32),
        mesh=mesh,
        scratch_types=[
            pltpu.VMEM((nb, K), jnp.int32),
            pltpu.VMEM((nb, K), jnp.int32),
            pltpu.VMEM((K, 16), jnp.float32),
            pltpu.VMEM((K, 16), jnp.float32),
            pltpu.VMEM((np_ // NS, 16), jnp.float32),
            pltpu.VMEM_SHARED((np_, 16), jnp.float32),
            pltpu.SemaphoreType.DMA,
            pltpu.SemaphoreType.DMA,
        ],
        compiler_params=params,
    )
    agg = pl.kernel(
        functools.partial(_agg_body, nn, h, nb),
        out_type=jax.ShapeDtypeStruct((NC, nn, h), jnp.float32),
        mesh=mesh,
        scratch_types=[
            pltpu.VMEM((nb, K), jnp.int32),
            pltpu.VMEM((nb, K), jnp.int32),
            pltpu.VMEM((K, h), jnp.float32),
            pltpu.VMEM((K, h), jnp.float32),
            pltpu.VMEM((K, h), jnp.float32),
            pltpu.VMEM_SHARED((nn, h), jnp.float32),
            pltpu.SemaphoreType.DMA,
            pltpu.SemaphoreType.DMA,
            pltpu.SemaphoreType.DMA,
        ],
        compiler_params=params,
    )
    return deg, agg


def _norm(col):
    return jnp.where(col > 0, lax.rsqrt(col), 0.0)


def _tc1_body(x_ref, degp_ref, w_ref, y_ref):
    dp = degp_ref[...]
    ns = _norm(dp[0, :, 0:1] + dp[1, :, 0:1])
    y_ref[...] = jnp.dot(x_ref[...], w_ref[...],
                         preferred_element_type=jnp.float32) * ns


def _tc2_body(aggp_ref, degp_ref, b_ref, w_ref, y_ref):
    dp = degp_ref[...]
    nd = _norm(dp[0, :, 1:2] + dp[1, :, 1:2])
    ns = _norm(dp[0, :, 0:1] + dp[1, :, 0:1])
    agg = aggp_ref[0, :, :] + aggp_ref[1, :, :]
    hcur = jnp.maximum(agg * nd + b_ref[...], 0.0)
    y_ref[...] = jnp.dot(hcur, w_ref[...],
                         preferred_element_type=jnp.float32) * ns


def _tc3_body(aggp_ref, degp_ref, b_ref, w_ref, bfc_ref, out_ref):
    dp = degp_ref[...]
    nd = _norm(dp[0, :, 1:2] + dp[1, :, 1:2])
    agg = aggp_ref[0, :, :] + aggp_ref[1, :, :]
    hcur = jnp.maximum(agg * nd + b_ref[...], 0.0)
    out_ref[...] = jnp.dot(hcur, w_ref[...],
                           preferred_element_type=jnp.float32) + bfc_ref[...]


def _row_block(rb, width):
    return pl.BlockSpec((rb, width), lambda i: (i, 0))


def _degp_block(rb):
    return pl.BlockSpec((NC, rb, 16), lambda i: (0, i, 0))


def _full(shape):
    ndim = len(shape)
    return pl.BlockSpec(shape, lambda i: (0,) * ndim)


def kernel(x, edge_index, W1, b1, W2, b2, Wfc, bfc):
    n, d = x.shape
    h = W1.shape[1]
    c = Wfc.shape[1]
    e = edge_index.shape[1]
    rb = 1000
    grid = (n // rb,)

    np_ = ((n + NS * 8 - 1) // (NS * 8)) * (NS * 8)

    e3 = edge_index.astype(jnp.int32).reshape(2, NW, e // (NW * K), K)
    ones_np_s = np.zeros((K, 16), np.float32)
    ones_np_s[:, 0] = 1.0
    ones_np_d = np.zeros((K, 16), np.float32)
    ones_np_d[:, 1] = 1.0
    ones_s = jnp.asarray(ones_np_s)
    ones_d = jnp.asarray(ones_np_d)
    zeros8 = jnp.asarray(np.zeros((np_ // NS, 16), np.float32))
    zerosh = jnp.asarray(np.zeros((np_ // NS // 8, h), np.float32))

    deg_k, agg_k = _make_sc_kernels(np_, n, h, e)
    degp = deg_k(e3, ones_s, ones_d, zeros8)

    tc1 = pl.pallas_call(
        _tc1_body,
        grid=grid,
        in_specs=[_row_block(rb, d), _degp_block(rb), _full((d, h))],
        out_specs=_row_block(rb, h),
        out_shape=jax.ShapeDtypeStruct((n, h), jnp.float32),
    )
    y1 = tc1(x, degp, W1)
    aggp1 = agg_k(y1, e3, zerosh)

    tc2 = pl.pallas_call(
        _tc2_body,
        grid=grid,
        in_specs=[pl.BlockSpec((NC, rb, h), lambda i: (0, i, 0)),
                  _degp_block(rb), _full((1, h)), _full((h, h))],
        out_specs=_row_block(rb, h),
        out_shape=jax.ShapeDtypeStruct((n, h), jnp.float32),
    )
    y2 = tc2(aggp1, degp, b1.reshape(1, h), W2)
    aggp2 = agg_k(y2, e3, zerosh)

    tc3 = pl.pallas_call(
        _tc3_body,
        grid=grid,
        in_specs=[pl.BlockSpec((NC, rb, h), lambda i: (0, i, 0)),
                  _degp_block(rb), _full((1, h)), _full((h, c)),
                  _full((1, c))],
        out_specs=_row_block(rb, c),
        out_shape=jax.ShapeDtypeStruct((n, c), jnp.float32),
    )
    return tc3(aggp2, degp, b2.reshape(1, h), Wfc, bfc.reshape(1, c))
